# baseline (speedup 1.0000x reference)
import jax
import jax.numpy as jnp
from jax.experimental import pallas as pl

B_BLK = 512
G = 4
K = 512


def _dot(a, b):
    return jax.lax.dot_general(
        a, b, (((1,), (0,)), ((), ())),
        precision=jax.lax.Precision.HIGHEST,
        preferred_element_type=jnp.float32)


def _dot_fast(a, b):
    return jax.lax.dot_general(
        a.astype(jnp.bfloat16), b.astype(jnp.bfloat16),
        (((1,), (0,)), ((), ())),
        preferred_element_type=jnp.float32)


def _select(onehot, planes, d):
    sel = _dot_fast(onehot, planes)
    return sel[:, :d] + (sel[:, d:2 * d] + sel[:, 2 * d:])


def _vqvae_body(x_ref, ew1, eb1, ew2, eb2, ew3, eb3,
                dw1, db1, dw2, db2, dw3, db3, cbp_ref,
                cbt_ref, code_ref, loss_ref):
    i = pl.program_id(0)
    x = x_ref[...]
    h = jnp.maximum(_dot_fast(x, ew1[...]) + eb1[...], 0.0)
    h = jnp.maximum(_dot_fast(h, ew2[...]) + eb2[...], 0.0)
    z = _dot_fast(h, ew3[...]) + eb3[...]

    blk = x.shape[0]
    half = blk // 2
    lane_iota = jax.lax.broadcasted_iota(jnp.int32, (half, K), 1).astype(jnp.float32)
    norms_all = [jnp.sum(cbt_ref[g] * cbt_ref[g], axis=0, keepdims=True)
                 for g in range(G)]
    dmin_sums = [jnp.float32(0.0)] * G
    quants = []
    for hh in range(2):
        resid = z[hh * half:(hh + 1) * half]
        quant = jnp.zeros_like(resid)
        for g in range(G):
            cbt = cbt_ref[g]
            rn = jnp.sum(resid * resid, axis=1, keepdims=True)
            d = rn - 2.0 * _dot_fast(resid, cbt) + norms_all[g]
            m = jnp.min(d, axis=1, keepdims=True)
            idxf = jnp.min(jnp.where(d == m, lane_iota, float(K)), axis=1,
                           keepdims=True)
            onehot = (lane_iota == idxf).astype(jnp.bfloat16)
            q = _select(onehot, cbp_ref[g], z.shape[1])
            quant = quant + q
            resid = resid - q
            code_ref[hh * half:(hh + 1) * half, g:g + 1] = idxf.astype(jnp.int32)
            dmin_sums[g] = dmin_sums[g] + jnp.sum(m)
        quants.append(quant)
    quant = jnp.concatenate(quants, axis=0)

    y = jnp.maximum(_dot_fast(quant, dw1[...]) + db1[...], 0.0)
    y = jnp.maximum(_dot_fast(y, dw2[...]) + db2[...], 0.0)
    dec = _dot_fast(y, dw3[...]) + db3[...]
    diff = x - dec
    rows = [jnp.sum(jnp.abs(diff)), jnp.sum(diff * diff)] + dmin_sums
    part = jnp.concatenate(
        [jnp.full((1, 128), r, jnp.float32) for r in rows], axis=0)

    @pl.when(i == 0)
    def _():
        loss_ref[...] = part

    @pl.when(i != 0)
    def _():
        loss_ref[...] += part


def kernel(state, enc_W1, enc_b1, enc_W2, enc_b2, enc_W3, enc_b3,
           dec_W1, dec_b1, dec_W2, dec_b2, dec_W3, dec_b3, codebooks):
    b = state.shape[0]
    x = state.reshape(b, -1)
    d_in = x.shape[1]
    h = enc_W1.shape[1]
    d = enc_W3.shape[1]
    cbt = codebooks.transpose(0, 2, 1)
    def _trunc_bf16(v):
        bits = jax.lax.bitcast_convert_type(v, jnp.uint32)
        return jax.lax.bitcast_convert_type(
            bits & jnp.uint32(0xFFFF0000), jnp.float32)

    cb_hi = _trunc_bf16(codebooks)
    r1 = codebooks - cb_hi
    cb_mid = _trunc_bf16(r1)
    cb_lo = r1 - cb_mid
    cb_planes = jnp.concatenate(
        [cb_hi.astype(jnp.bfloat16), cb_mid.astype(jnp.bfloat16),
         cb_lo.astype(jnp.bfloat16)], axis=-1)

    grid = b // B_BLK
    full = lambda shp: pl.BlockSpec(shp, lambda i, _s=None: tuple(0 for _ in shp))
    codes, losses = pl.pallas_call(
        _vqvae_body,
        grid=(grid,),
        in_specs=[
            pl.BlockSpec((B_BLK, d_in), lambda i: (i, 0)),
            full((d_in, h)), full((1, h)),
            full((h, h)), full((1, h)),
            full((h, d)), full((1, d)),
            full((d, h)), full((1, h)),
            full((h, h)), full((1, h)),
            full((h, d_in)), full((1, d_in)),
            full((G, K, 3 * d)), full((G, d, K)),
        ],
        out_specs=[
            pl.BlockSpec((B_BLK, G), lambda i: (i, 0)),
            pl.BlockSpec((6, 128), lambda i: (0, 0)),
        ],
        out_shape=[
            jax.ShapeDtypeStruct((b, G), jnp.int32),
            jax.ShapeDtypeStruct((6, 128), jnp.float32),
        ],
    )(x, enc_W1, enc_b1.reshape(1, h), enc_W2, enc_b2.reshape(1, h),
      enc_W3, enc_b3.reshape(1, d), dec_W1, dec_b1.reshape(1, h),
      dec_W2, dec_b2.reshape(1, h), dec_W3, dec_b3.reshape(1, d_in),
      cb_planes, cbt)

    sums = losses[:, 0]
    encoder_loss = sums[0] / (b * d_in)
    vqvae_recon_loss = sums[1] / (b * d_in)
    vq_loss_sum = jnp.sum(sums[2:2 + G]) / (b * d)
    loss = encoder_loss * 1.0 + vq_loss_sum * 5.0
    return (loss, codes, vq_loss_sum, vqvae_recon_loss, encoder_loss)


# speedup vs baseline: 1.2325x; 1.2325x over previous
import jax
import jax.numpy as jnp
from jax.experimental import pallas as pl

B_BLK = 1024
G = 4
K = 512


def _dot(a, b):
    return jax.lax.dot_general(
        a, b, (((1,), (0,)), ((), ())),
        precision=jax.lax.Precision.HIGHEST,
        preferred_element_type=jnp.float32)


def _dot_fast(a, b):
    return jax.lax.dot_general(
        a.astype(jnp.bfloat16), b.astype(jnp.bfloat16),
        (((1,), (0,)), ((), ())),
        preferred_element_type=jnp.float32)


def _select(onehot, planes, d):
    sel = _dot_fast(onehot, planes)
    return sel[:, :d] + (sel[:, d:2 * d] + sel[:, 2 * d:])


def _vqvae_body(x_ref, ew1, eb1, ew2, eb2, ew3, eb3,
                dw1, db1, dw2, db2, dw3, db3, cbp_ref,
                cbt_ref, cbtn_ref, code_ref, loss_ref):
    i = pl.program_id(0)
    x = x_ref[...]
    h = jnp.maximum(_dot_fast(x, ew1[...]) + eb1[...], 0.0)
    h = jnp.maximum(_dot_fast(h, ew2[...]) + eb2[...], 0.0)
    z = _dot_fast(h, ew3[...]) + eb3[...]

    blk = x.shape[0]
    lane_iota = jax.lax.broadcasted_iota(jnp.int32, (blk, K), 1).astype(jnp.float32)
    resid = z
    quant = jnp.zeros_like(z)
    dmin_sums = []
    for g in range(G):
        cbt = cbt_ref[g]
        norms = jnp.sum(cbt * cbt, axis=0, keepdims=True)
        rn = jnp.sum(resid * resid, axis=1, keepdims=True)
        s2 = _dot_fast(resid, cbtn_ref[g])
        d = rn + s2 + norms
        m = jnp.min(d, axis=1, keepdims=True)
        idxf = jnp.min(jnp.where(d == m, lane_iota, float(K)), axis=1,
                       keepdims=True)
        onehot = (lane_iota == idxf).astype(jnp.bfloat16)
        q = _select(onehot, cbp_ref[g], z.shape[1])
        quant = quant + q
        resid = resid - q
        code_ref[:, g:g + 1] = idxf.astype(jnp.int32)
        dmin_sums.append(jnp.sum(m))

    y = jnp.maximum(_dot_fast(quant, dw1[...]) + db1[...], 0.0)
    y = jnp.maximum(_dot_fast(y, dw2[...]) + db2[...], 0.0)
    dec = _dot_fast(y, dw3[...]) + db3[...]
    diff = x - dec
    rows = [jnp.sum(jnp.abs(diff)), jnp.sum(diff * diff)] + dmin_sums
    part = jnp.concatenate(
        [jnp.full((1, 128), r, jnp.float32) for r in rows], axis=0)

    @pl.when(i == 0)
    def _():
        loss_ref[...] = part

    @pl.when(i != 0)
    def _():
        loss_ref[...] += part


def kernel(state, enc_W1, enc_b1, enc_W2, enc_b2, enc_W3, enc_b3,
           dec_W1, dec_b1, dec_W2, dec_b2, dec_W3, dec_b3, codebooks):
    b = state.shape[0]
    x = state.reshape(b, -1)
    d_in = x.shape[1]
    h = enc_W1.shape[1]
    d = enc_W3.shape[1]
    cbt = codebooks.transpose(0, 2, 1)
    def _trunc_bf16(v):
        bits = jax.lax.bitcast_convert_type(v, jnp.uint32)
        return jax.lax.bitcast_convert_type(
            bits & jnp.uint32(0xFFFF0000), jnp.float32)

    cb_hi = _trunc_bf16(codebooks)
    r1 = codebooks - cb_hi
    cb_mid = _trunc_bf16(r1)
    cb_lo = r1 - cb_mid
    cb_planes = jnp.concatenate(
        [cb_hi.astype(jnp.bfloat16), cb_mid.astype(jnp.bfloat16),
         cb_lo.astype(jnp.bfloat16)], axis=-1)

    grid = b // B_BLK
    full = lambda shp: pl.BlockSpec(shp, lambda i, _s=None: tuple(0 for _ in shp))
    codes, losses = pl.pallas_call(
        _vqvae_body,
        grid=(grid,),
        in_specs=[
            pl.BlockSpec((B_BLK, d_in), lambda i: (i, 0)),
            full((d_in, h)), full((1, h)),
            full((h, h)), full((1, h)),
            full((h, d)), full((1, d)),
            full((d, h)), full((1, h)),
            full((h, h)), full((1, h)),
            full((h, d_in)), full((1, d_in)),
            full((G, K, 3 * d)), full((G, d, K)), full((G, d, K)),
        ],
        out_specs=[
            pl.BlockSpec((B_BLK, G), lambda i: (i, 0)),
            pl.BlockSpec((6, 128), lambda i: (0, 0)),
        ],
        out_shape=[
            jax.ShapeDtypeStruct((b, G), jnp.int32),
            jax.ShapeDtypeStruct((6, 128), jnp.float32),
        ],
    )(x, enc_W1.astype(jnp.bfloat16), enc_b1.reshape(1, h),
      enc_W2.astype(jnp.bfloat16), enc_b2.reshape(1, h),
      enc_W3.astype(jnp.bfloat16), enc_b3.reshape(1, d),
      dec_W1.astype(jnp.bfloat16), dec_b1.reshape(1, h),
      dec_W2.astype(jnp.bfloat16), dec_b2.reshape(1, h),
      dec_W3.astype(jnp.bfloat16), dec_b3.reshape(1, d_in),
      cb_planes, cbt, (cbt * -2.0).astype(jnp.bfloat16))

    sums = losses[:, 0]
    encoder_loss = sums[0] / (b * d_in)
    vqvae_recon_loss = sums[1] / (b * d_in)
    vq_loss_sum = jnp.sum(sums[2:2 + G]) / (b * d)
    loss = encoder_loss * 1.0 + vq_loss_sum * 5.0
    return (loss, codes, vq_loss_sum, vqvae_recon_loss, encoder_loss)
